# baseline (speedup 1.0000x reference)
import jax
import jax.numpy as jnp
from jax.experimental import pallas as pl
from jax.experimental.pallas import tpu as pltpu

D_MODEL = 1024
D_FF = 2816
NUM_EXPERTS = 16
N_TOK = 32
F_BLK = 256
NF = D_FF // F_BLK


def _moe_kernel(xT_ref, wr_ref, w1_ref, w3_ref, w2_ref, out_ref, wT_ref):
    e = pl.program_id(0)
    f = pl.program_id(1)

    @pl.when(jnp.logical_and(e == 0, f == 0))
    def _router():
        xT = xT_ref[...]
        lT = jnp.dot(wr_ref[...], xT,
                     preferred_element_type=jnp.float32)
        m = jnp.max(lT, axis=0, keepdims=True)
        ex = jnp.exp(lT - m)
        p = ex / jnp.sum(ex, axis=0, keepdims=True)
        iota_e = jax.lax.broadcasted_iota(jnp.int32, (NUM_EXPERTS, N_TOK), 0)
        m1 = jnp.max(p, axis=0, keepdims=True)
        i1 = jnp.min(jnp.where(p == m1, iota_e, NUM_EXPERTS),
                     axis=0, keepdims=True)
        first = iota_e == i1
        pm = jnp.where(first, -1.0, p)
        m2 = jnp.max(pm, axis=0, keepdims=True)
        i2 = jnp.min(jnp.where(pm == m2, iota_e, NUM_EXPERTS),
                     axis=0, keepdims=True)
        second = iota_e == i2
        keep = jnp.logical_or(first, second)
        denom = m1 + m2 + 1e-9
        wT_ref[...] = jnp.where(keep, p, 0.0) / denom
        out_ref[...] = jnp.zeros_like(out_ref)

    xT = xT_ref[...]
    h1 = jnp.dot(w1_ref[0], xT, preferred_element_type=jnp.float32)
    h3 = jnp.dot(w3_ref[0], xT, preferred_element_type=jnp.float32)
    h = (h1 * jax.nn.sigmoid(h1)) * h3
    sel = jax.lax.broadcasted_iota(jnp.int32, (NUM_EXPERTS, 1), 0) == e
    wrow = jnp.sum(jnp.where(sel, wT_ref[...], 0.0),
                   axis=0, keepdims=True)
    out_ref[...] += jnp.dot(w2_ref[0], h * wrow,
                            preferred_element_type=jnp.float32)


def kernel(x, W_router, W1, W3, W2):
    b, s, d = x.shape
    n = b * s
    xT = x.reshape(n, d).T
    out_t = pl.pallas_call(
        _moe_kernel,
        grid=(NUM_EXPERTS, NF),
        in_specs=[
            pl.BlockSpec((d, n), lambda e, f: (0, 0)),
            pl.BlockSpec((NUM_EXPERTS, d), lambda e, f: (0, 0)),
            pl.BlockSpec((1, F_BLK, d), lambda e, f: (e, f, 0)),
            pl.BlockSpec((1, F_BLK, d), lambda e, f: (e, f, 0)),
            pl.BlockSpec((1, d, F_BLK), lambda e, f: (e, 0, f)),
        ],
        out_specs=pl.BlockSpec((d, n), lambda e, f: (0, 0)),
        out_shape=jax.ShapeDtypeStruct((d, n), jnp.float32),
        scratch_shapes=[pltpu.VMEM((NUM_EXPERTS, n), jnp.float32)],
    )(xT, W_router, W1, W3, W2)
    return out_t.T.reshape(b, s, d)


# speedup vs baseline: 1.2844x; 1.2844x over previous
import functools

import jax
import jax.numpy as jnp
from jax.experimental import pallas as pl
from jax.experimental.pallas import tpu as pltpu
from jax.experimental.pallas import tpu_sc as plsc

D_MODEL = 1024
D_FF = 2816
NUM_EXPERTS = 16
N_TOK = 32
F_BLK = 1408
NF = D_FF // F_BLK

_MESH = plsc.VectorSubcoreMesh(core_axis_name="c", subcore_axis_name="s")


@functools.partial(
    pl.kernel,
    mesh=_MESH,
    out_type=jax.ShapeDtypeStruct((N_TOK, NUM_EXPERTS), jnp.float32),
    scratch_types=[
        pltpu.VMEM((D_MODEL,), jnp.float32),
        pltpu.VMEM((D_MODEL * NUM_EXPERTS,), jnp.float32),
        pltpu.VMEM((NUM_EXPERTS,), jnp.float32),
    ],
)
def _sc_router(x_hbm, wrT_hbm, out_hbm, x_v, wr_v, w_v):
    t = jax.lax.axis_index("s") * 2 + jax.lax.axis_index("c")
    pltpu.sync_copy(x_hbm.at[t], x_v)
    iota = jax.lax.broadcasted_iota(jnp.int32, (NUM_EXPERTS,), 0)

    def shuf(v, k):
        return v.at[iota ^ k].get(mode="promise_in_bounds")

    def allreduce(v, op):
        for k in (1, 2, 4, 8):
            v = op(v, shuf(v, k))
        return v

    pltpu.sync_copy(wrT_hbm, wr_v)
    logit = jnp.zeros((NUM_EXPERTS,), jnp.float32)
    for jc in range(D_MODEL // 16):
        xc = x_v[pl.ds(jc * 16, 16)]
        for i in range(16):
            xi = xc.at[jnp.full((16,), i, jnp.int32)].get(
                mode="promise_in_bounds")
            logit = logit + xi * wr_v[pl.ds((jc * 16 + i) * 16, 16)]
    m = allreduce(logit, jnp.maximum)
    ex = jnp.exp(logit - m)
    p = ex / allreduce(ex, jnp.add)
    m1 = allreduce(p, jnp.maximum)
    i1 = allreduce(jnp.where(p == m1, iota, NUM_EXPERTS), jnp.minimum)
    first = iota == i1
    pm = jnp.where(first, -1.0, p)
    m2 = allreduce(pm, jnp.maximum)
    i2 = allreduce(jnp.where(pm == m2, iota, NUM_EXPERTS), jnp.minimum)
    second = iota == i2
    keep = jnp.logical_or(first, second)
    w_v[...] = jnp.where(keep, p, 0.0) / (m1 + m2 + 1e-9)
    pltpu.sync_copy(w_v, out_hbm.at[t])


def _moe_kernel(xT_ref, w_ref, w1_ref, w3_ref, w2_ref, out_ref, wT_ref):
    e = pl.program_id(0)
    f = pl.program_id(1)

    @pl.when(jnp.logical_and(e == 0, f == 0))
    def _prologue():
        wT_ref[...] = w_ref[...].T
        out_ref[...] = jnp.zeros_like(out_ref)

    xT = xT_ref[...]
    h1 = jnp.dot(w1_ref[0], xT, preferred_element_type=jnp.float32)
    h3 = jnp.dot(w3_ref[0], xT, preferred_element_type=jnp.float32)
    h = (h1 * jax.nn.sigmoid(h1)) * h3
    sel = jax.lax.broadcasted_iota(jnp.int32, (NUM_EXPERTS, 1), 0) == e
    wrow = jnp.sum(jnp.where(sel, wT_ref[...], 0.0),
                   axis=0, keepdims=True)
    out_ref[...] += jnp.dot(w2_ref[0], h * wrow,
                            preferred_element_type=jnp.float32)


def kernel(x, W_router, W1, W3, W2):
    b, s, d = x.shape
    n = b * s
    x_flat = x.reshape(n, d)
    w_mix = _sc_router(x_flat, W_router.T.reshape(-1))
    xT = x_flat.T
    out_t = pl.pallas_call(
        _moe_kernel,
        grid=(NUM_EXPERTS, NF),
        in_specs=[
            pl.BlockSpec((d, n), lambda e, f: (0, 0)),
            pl.BlockSpec((n, NUM_EXPERTS), lambda e, f: (0, 0)),
            pl.BlockSpec((1, F_BLK, d), lambda e, f: (e, f, 0)),
            pl.BlockSpec((1, F_BLK, d), lambda e, f: (e, f, 0)),
            pl.BlockSpec((1, d, F_BLK), lambda e, f: (e, 0, f)),
        ],
        out_specs=pl.BlockSpec((d, n), lambda e, f: (0, 0)),
        out_shape=jax.ShapeDtypeStruct((d, n), jnp.float32),
        scratch_shapes=[pltpu.VMEM((NUM_EXPERTS, n), jnp.float32)],
    )(xT, w_mix, W1, W3, W2)
    return out_t.T.reshape(b, s, d)


# speedup vs baseline: 1.4263x; 1.1104x over previous
import jax
import jax.numpy as jnp
from jax.experimental import pallas as pl
from jax.experimental.pallas import tpu as pltpu

D_MODEL = 1024
D_FF = 2816
NUM_EXPERTS = 16
N_TOK = 32
F_BLK = 1408
NF = D_FF // F_BLK


def _moe_kernel(xT_ref, wr_ref, w1_ref, w3_ref, w2_ref, out_ref, wT_ref,
                h_ref):
    e = pl.program_id(0)
    f = pl.program_id(1)

    @pl.when(jnp.logical_and(e == 0, f == 0))
    def _router():
        xT = xT_ref[...]
        lT = jnp.dot(wr_ref[...], xT,
                     preferred_element_type=jnp.float32)
        m = jnp.max(lT, axis=0, keepdims=True)
        ex = jnp.exp(lT - m)
        p = ex / jnp.sum(ex, axis=0, keepdims=True)
        iota_e = jax.lax.broadcasted_iota(jnp.int32, (NUM_EXPERTS, N_TOK), 0)
        m1 = jnp.max(p, axis=0, keepdims=True)
        i1 = jnp.min(jnp.where(p == m1, iota_e, NUM_EXPERTS),
                     axis=0, keepdims=True)
        first = iota_e == i1
        pm = jnp.where(first, -1.0, p)
        m2 = jnp.max(pm, axis=0, keepdims=True)
        i2 = jnp.min(jnp.where(pm == m2, iota_e, NUM_EXPERTS),
                     axis=0, keepdims=True)
        second = iota_e == i2
        keep = jnp.logical_or(first, second)
        denom = m1 + m2 + 1e-9
        wT_ref[...] = jnp.where(keep, p, 0.0) / denom
        out_ref[...] = jnp.zeros_like(out_ref)

    xT = xT_ref[...]
    h1 = jnp.dot(w1_ref[0], xT, preferred_element_type=jnp.float32)
    h3 = jnp.dot(w3_ref[0], xT, preferred_element_type=jnp.float32)
    h = (h1 * jax.nn.sigmoid(h1)) * h3
    sel = jax.lax.broadcasted_iota(jnp.int32, (NUM_EXPERTS, 1), 0) == e
    wrow = jnp.sum(jnp.where(sel, wT_ref[...], 0.0),
                   axis=0, keepdims=True)
    h_ref[pl.ds(f * F_BLK, F_BLK), :] = h * wrow

    @pl.when(f == NF - 1)
    def _expert_out():
        out_ref[...] += jnp.dot(w2_ref[0], h_ref[...],
                                preferred_element_type=jnp.float32)


def kernel(x, W_router, W1, W3, W2):
    b, s, d = x.shape
    n = b * s
    xT = x.reshape(n, d).T
    out_t = pl.pallas_call(
        _moe_kernel,
        grid=(NUM_EXPERTS, NF),
        in_specs=[
            pl.BlockSpec((d, n), lambda e, f: (0, 0)),
            pl.BlockSpec((NUM_EXPERTS, d), lambda e, f: (0, 0)),
            pl.BlockSpec((1, F_BLK, d), lambda e, f: (e, f, 0)),
            pl.BlockSpec((1, F_BLK, d), lambda e, f: (e, f, 0)),
            pl.BlockSpec((1, d, D_FF), lambda e, f: (e, 0, 0)),
        ],
        out_specs=pl.BlockSpec((d, n), lambda e, f: (0, 0)),
        out_shape=jax.ShapeDtypeStruct((d, n), jnp.float32),
        scratch_shapes=[pltpu.VMEM((NUM_EXPERTS, n), jnp.float32),
                        pltpu.VMEM((D_FF, n), jnp.float32)],
    )(xT, W_router, W1, W3, W2)
    return out_t.T.reshape(b, s, d)


# speedup vs baseline: 1.4926x; 1.0465x over previous
import jax
import jax.numpy as jnp
from jax.experimental import pallas as pl
from jax.experimental.pallas import tpu as pltpu

D_MODEL = 1024
D_FF = 2816
NUM_EXPERTS = 16
N_TOK = 32
F_BLK = 1408
NF = D_FF // F_BLK


def _moe_kernel(xT_ref, wr_ref, w1_ref, w3_ref, w2_ref, out_ref, wT_ref):
    e = pl.program_id(0)
    f = pl.program_id(1)

    @pl.when(jnp.logical_and(e == 0, f == 0))
    def _router():
        xT = xT_ref[...]
        lT = jnp.dot(wr_ref[...], xT,
                     preferred_element_type=jnp.float32)
        m = jnp.max(lT, axis=0, keepdims=True)
        ex = jnp.exp(lT - m)
        p = ex / jnp.sum(ex, axis=0, keepdims=True)
        iota_e = jax.lax.broadcasted_iota(jnp.int32, (NUM_EXPERTS, N_TOK), 0)
        m1 = jnp.max(p, axis=0, keepdims=True)
        i1 = jnp.min(jnp.where(p == m1, iota_e, NUM_EXPERTS),
                     axis=0, keepdims=True)
        first = iota_e == i1
        pm = jnp.where(first, -1.0, p)
        m2 = jnp.max(pm, axis=0, keepdims=True)
        i2 = jnp.min(jnp.where(pm == m2, iota_e, NUM_EXPERTS),
                     axis=0, keepdims=True)
        second = iota_e == i2
        keep = jnp.logical_or(first, second)
        denom = m1 + m2 + 1e-9
        wT_ref[...] = jnp.where(keep, p, 0.0) / denom
        out_ref[...] = jnp.zeros_like(out_ref)

    xT = xT_ref[...]
    h1 = jnp.dot(w1_ref[0], xT, preferred_element_type=jnp.float32)
    h3 = jnp.dot(w3_ref[0], xT, preferred_element_type=jnp.float32)
    h = (h1 * jax.nn.sigmoid(h1)) * h3
    sel = jax.lax.broadcasted_iota(jnp.int32, (NUM_EXPERTS, 1), 0) == e
    wrow = jnp.sum(jnp.where(sel, wT_ref[...], 0.0),
                   axis=0, keepdims=True)
    out_ref[...] += jnp.dot(w2_ref[0], h * wrow,
                            preferred_element_type=jnp.float32)


def kernel(x, W_router, W1, W3, W2):
    b, s, d = x.shape
    n = b * s
    xT = x.reshape(n, d).T
    out_t = pl.pallas_call(
        _moe_kernel,
        grid=(NUM_EXPERTS, NF),
        in_specs=[
            pl.BlockSpec((d, n), lambda e, f: (0, 0)),
            pl.BlockSpec((NUM_EXPERTS, d), lambda e, f: (0, 0)),
            pl.BlockSpec((1, F_BLK, d), lambda e, f: (e, f, 0)),
            pl.BlockSpec((1, F_BLK, d), lambda e, f: (e, f, 0)),
            pl.BlockSpec((1, d, F_BLK), lambda e, f: (e, 0, f)),
        ],
        out_specs=pl.BlockSpec((d, n), lambda e, f: (0, 0)),
        out_shape=jax.ShapeDtypeStruct((d, n), jnp.float32),
        scratch_shapes=[pltpu.VMEM((NUM_EXPERTS, n), jnp.float32)],
    )(xT, W_router, W1, W3, W2)
    return out_t.T.reshape(b, s, d)
